# baseline (speedup 1.0000x reference)
import functools

import jax
import jax.numpy as jnp
from jax import lax
from jax.experimental import pallas as pl
from jax.experimental.pallas import tpu as pltpu
from jax.experimental.pallas import tpu_sc as plsc

N = 10000
D = 128
E = 320000

NC = 2
NS = 16
NW = NC * NS
NP = 10016
RPT = NP // NS
ZR = 64

_mesh = plsc.VectorSubcoreMesh(
    core_axis_name="c", subcore_axis_name="s", num_cores=NC, num_subcores=NS)


def _zero_rows16(ref, nrows):
    zv = jnp.zeros((16,), jnp.float32)

    def body(r, carry):
        ref[r] = zv
        return carry

    lax.fori_loop(0, nrows, body, 0)


def _make_sc_segsum(with_cnt, ch, cpw, nstages):
    spc = cpw // nstages
    out_type = [jax.ShapeDtypeStruct((NC, NP, D), jnp.float32)]
    scratch = [
        pltpu.VMEM((spc, ch), jnp.int32),
        pltpu.VMEM((spc, ch), jnp.int32),
        pltpu.VMEM((2, ch, D), jnp.float32),
        pltpu.VMEM_SHARED((NP, D), jnp.float32),
        pltpu.SemaphoreType.DMA,
        pltpu.SemaphoreType.DMA,
    ]
    if with_cnt:
        out_type.append(jax.ShapeDtypeStruct((NC, NP, 16), jnp.float32))
        scratch += [
            pltpu.VMEM((ZR, 16), jnp.float32),
            pltpu.VMEM((ch, 16), jnp.float32),
            pltpu.VMEM_SHARED((NP, 16), jnp.float32),
        ]

    @functools.partial(
        pl.kernel, out_type=tuple(out_type), mesh=_mesh,
        scratch_types=tuple(scratch),
        compiler_params=pltpu.CompilerParams(use_tc_tiling_on_sc=False))
    def sc_segsum(x_hbm, edges_hbm, *rest):
        if with_cnt:
            (out_hbm, cnt_hbm, src_v, dst_v, rows_v, acc_sh,
             sem0, sem1, zcnt, ones_v, cnt_sh) = rest
        else:
            (out_hbm, src_v, dst_v, rows_v, acc_sh, sem0, sem1) = rest

        core = lax.axis_index("c")
        sub = lax.axis_index("s")
        wid = sub * NC + core

        zv = jnp.zeros((16,), jnp.float32)

        def zrow(r, carry):
            def zcol(cc, carry2):
                rows_v[0, r, pl.ds(cc * 16, 16)] = zv
                return carry2
            return lax.fori_loop(0, D // 16, zcol, carry)

        lax.fori_loop(0, ZR, zrow, 0)
        if with_cnt:
            _zero_rows16(zcnt, ZR)
            ov = jnp.ones((16,), jnp.float32)

            def orow(r, carry):
                ones_v[r] = ov
                return carry

            lax.fori_loop(0, ch, orow, 0)

        base = sub * RPT
        offs = [k * ZR for k in range(RPT // ZR)]
        if RPT % ZR:
            offs.append(RPT - ZR)
        zrows = rows_v.at[0, pl.ds(0, ZR)]
        for off in offs:
            pltpu.sync_copy(zrows, acc_sh.at[pl.ds(base + off, ZR)])
            if with_cnt:
                pltpu.sync_copy(zcnt, cnt_sh.at[pl.ds(base + off, ZR)])
        plsc.subcore_barrier()

        sems = (sem0, sem1)

        def chunk(ci, buf):
            pltpu.make_async_copy(
                x_hbm.at[src_v.at[ci]], rows_v.at[buf], sems[buf]).wait()

            @pl.when(ci + 1 < spc)
            def _():
                pltpu.async_copy(
                    x_hbm.at[src_v.at[ci + 1]], rows_v.at[1 - buf],
                    sems[1 - buf])

            pltpu.sync_copy(rows_v.at[buf], acc_sh.at[dst_v.at[ci]], add=True)
            if with_cnt:
                pltpu.sync_copy(ones_v, cnt_sh.at[dst_v.at[ci]], add=True)

        def group(g, carry):
            chunk(2 * g, 0)
            chunk(2 * g + 1, 1)
            return carry

        for s in range(nstages):
            pltpu.sync_copy(edges_hbm.at[0, wid, pl.ds(s * spc, spc)], src_v)
            pltpu.sync_copy(edges_hbm.at[1, wid, pl.ds(s * spc, spc)], dst_v)
            pltpu.async_copy(x_hbm.at[src_v.at[0]], rows_v.at[0], sem0)
            lax.fori_loop(0, spc // 2, group, 0)
        plsc.subcore_barrier()

        pltpu.sync_copy(acc_sh.at[pl.ds(base, RPT)],
                        out_hbm.at[core, pl.ds(base, RPT)])
        if with_cnt:
            pltpu.sync_copy(cnt_sh.at[pl.ds(base, RPT)],
                            cnt_hbm.at[core, pl.ds(base, RPT)])

    return sc_segsum


CH0, CPW0, NST0 = 125, 80, 5
CH1, CPW1, NST1 = 125, 80, 2
_sc_segsum_cnt = _make_sc_segsum(True, CH0, CPW0, NST0)
_sc_segsum = _make_sc_segsum(False, CH1, CPW1, NST1)


_R = 1000


def _dot_t(a, w):
    return jax.lax.dot_general(
        a, w, (((1,), (1,)), ((), ())),
        preferred_element_type=jnp.float32)


def _mean_rows(p_ref, cnt_ref):
    i = pl.program_id(0)
    s = p_ref[0] + p_ref[1]
    c = (cnt_ref[0, pl.ds(i * _R, _R), 0:1]
         + cnt_ref[1, pl.ds(i * _R, _R), 0:1])
    return s / jnp.maximum(c, 1.0)


def _dense_ln_body(p_ref, cnt_ref, x_ref, wl_ref, wr_ref, bl_ref, g_ref,
                   b_ref, o_ref):
    mean = _mean_rows(p_ref, cnt_ref)
    x = x_ref[...]
    h = _dot_t(mean, wl_ref[...]) + _dot_t(x, wr_ref[...]) + bl_ref[...]
    m = jnp.mean(h, axis=-1, keepdims=True)
    v = jnp.mean((h - m) * (h - m), axis=-1, keepdims=True)
    hn = (h - m) / jnp.sqrt(v + 1e-5) * g_ref[...] + b_ref[...]
    o_ref[...] = jnp.maximum(hn, 0.0) + x


def _dense_plain_body(p_ref, cnt_ref, x_ref, wl_ref, wr_ref, bl_ref, o_ref):
    mean = _mean_rows(p_ref, cnt_ref)
    h = _dot_t(mean, wl_ref[...]) + _dot_t(x_ref[...], wr_ref[...])
    o_ref[...] = h + bl_ref[...]


def _dense(body, n_extra):
    in_specs = [
        pl.BlockSpec((NC, _R, D), lambda i: (0, i, 0)),
        pl.BlockSpec((NC, NP, 16), lambda i: (0, 0, 0)),
        pl.BlockSpec((_R, D), lambda i: (i, 0)),
        pl.BlockSpec((D, D), lambda i: (0, 0)),
        pl.BlockSpec((D, D), lambda i: (0, 0)),
        pl.BlockSpec((1, D), lambda i: (0, 0)),
    ]
    in_specs += [pl.BlockSpec((1, D), lambda i: (0, 0))] * n_extra
    return pl.pallas_call(
        body,
        grid=(N // _R,),
        in_specs=in_specs,
        out_specs=pl.BlockSpec((_R, D), lambda i: (i, 0)),
        out_shape=jax.ShapeDtypeStruct((N, D), jnp.float32),
    )


_dense_ln = _dense(_dense_ln_body, 2)
_dense_plain = _dense(_dense_plain_body, 0)


def kernel(x, edge_index, Wl0, bl0, Wr0, Wl1, bl1, Wr1, Wl2, bl2, Wr2,
           g0, b0, g1, b1):
    edges = edge_index.reshape(2, NW, CPW0, CH0)

    def r(a):
        return a.reshape(1, D)

    p, cnt = _sc_segsum_cnt(x, edges)
    x1 = _dense_ln(p, cnt, x, Wl0, Wr0, r(bl0), r(g0), r(b0))
    (p,) = _sc_segsum(x1, edges)
    x2 = _dense_ln(p, cnt, x1, Wl1, Wr1, r(bl1), r(g1), r(b1))
    (p,) = _sc_segsum(x2, edges)
    return _dense_plain(p, cnt, x2, Wl2, Wr2, r(bl2))
